# baseline (speedup 1.0000x reference)
import functools

import jax
import jax.numpy as jnp
from jax.experimental import pallas as pl

ROWS, COLS = 128, 100000
ROW_BLOCK = 16


def _threefry_bits_xor(flat_u32):
    k0 = jnp.uint32(0)
    k1 = jnp.uint32(1)
    ks2 = jnp.uint32(0x1BD11BDA) ^ k0 ^ k1
    rot1 = (13, 15, 26, 6)
    rot2 = (17, 29, 16, 24)

    def rotl(x, r):
        return (x << jnp.uint32(r)) | (x >> jnp.uint32(32 - r))

    x0 = jnp.zeros_like(flat_u32) + k0
    x1 = flat_u32 + k1
    inject = ((k1, ks2, 1), (ks2, k0, 2), (k0, k1, 3), (k1, ks2, 4),
              (ks2, k0, 5))
    for i in range(5):
        for r in (rot1 if i % 2 == 0 else rot2):
            x0 = x0 + x1
            x1 = rotl(x1, r)
            x1 = x1 ^ x0
        a, b, c = inject[i]
        x0 = x0 + a
        x1 = x1 + b + jnp.uint32(c)
    return x0 ^ x1


def _gumbel_softmax_block(logits_ref, out_ref):
    i = pl.program_id(0)
    shape = logits_ref.shape

    row = jax.lax.broadcasted_iota(jnp.uint32, shape, 0)
    col = jax.lax.broadcasted_iota(jnp.uint32, shape, 1)
    base = (jnp.uint32(i) * jnp.uint32(ROW_BLOCK)) * jnp.uint32(COLS)
    flat = base + row * jnp.uint32(COLS) + col

    bits = _threefry_bits_xor(flat)

    fbits = jax.lax.bitcast_convert_type(
        (bits >> jnp.uint32(9)) | jnp.uint32(0x3F800000), jnp.float32)
    minval = jnp.float32(1e-10)
    u = fbits - jnp.float32(1.0)
    u = u * (jnp.float32(1.0) - minval) + minval
    u = jnp.maximum(u, minval)

    noise = -jnp.log(-jnp.log(u))
    z = logits_ref[...] + noise

    m = jnp.max(z, axis=-1, keepdims=True)
    e = jnp.exp(z - m)
    s = jnp.sum(e, axis=-1, keepdims=True)
    out_ref[...] = e / s


@jax.jit
def kernel(logits):
    grid = (ROWS // ROW_BLOCK,)
    return pl.pallas_call(
        _gumbel_softmax_block,
        grid=grid,
        in_specs=[pl.BlockSpec((ROW_BLOCK, COLS), lambda i: (i, 0))],
        out_specs=pl.BlockSpec((ROW_BLOCK, COLS), lambda i: (i, 0)),
        out_shape=jax.ShapeDtypeStruct((ROWS, COLS), jnp.float32),
    )(logits)


# speedup vs baseline: 1.2177x; 1.2177x over previous
import jax
import jax.numpy as jnp
from jax.experimental import pallas as pl

ROWS, COLS = 128, 100000
ROW_BLOCK = 8
CHUNK = 2048
NFULL = COLS // CHUNK
TAIL = COLS - NFULL * CHUNK


def _threefry_bits_xor(flat_u32):
    k0 = jnp.uint32(0)
    k1 = jnp.uint32(1)
    ks2 = jnp.uint32(0x1BD11BDA) ^ k0 ^ k1
    rot1 = (13, 15, 26, 6)
    rot2 = (17, 29, 16, 24)

    def rotl(x, r):
        return (x << jnp.uint32(r)) | (x >> jnp.uint32(32 - r))

    x0 = jnp.zeros_like(flat_u32) + k0
    x1 = flat_u32 + k1
    inject = ((k1, ks2, 1), (ks2, k0, 2), (k0, k1, 3), (k1, ks2, 4),
              (ks2, k0, 5))
    for i in range(5):
        for r in (rot1 if i % 2 == 0 else rot2):
            x0 = x0 + x1
            x1 = rotl(x1, r)
            x1 = x1 ^ x0
        a, b, c = inject[i]
        x0 = x0 + a
        x1 = x1 + b + jnp.uint32(c)
    return x0 ^ x1


def _z_chunk(logits_chunk, flat_u32):
    bits = _threefry_bits_xor(flat_u32)
    fbits = jax.lax.bitcast_convert_type(
        (bits >> jnp.uint32(9)) | jnp.uint32(0x3F800000), jnp.float32)
    minval = jnp.float32(1e-10)
    u = fbits - jnp.float32(1.0)
    u = u * (jnp.float32(1.0) - minval) + minval
    u = jnp.maximum(u, minval)
    noise = -jnp.log(-jnp.log(u))
    return logits_chunk + noise


def _gumbel_softmax_block(logits_ref, out_ref):
    i = pl.program_id(0)
    base = jnp.uint32(i) * jnp.uint32(ROW_BLOCK * COLS)

    cshape = (ROW_BLOCK, CHUNK)
    rowcol = (jax.lax.broadcasted_iota(jnp.uint32, cshape, 0)
              * jnp.uint32(COLS)
              + jax.lax.broadcasted_iota(jnp.uint32, cshape, 1))
    tshape = (ROW_BLOCK, TAIL)
    rowcol_t = (jax.lax.broadcasted_iota(jnp.uint32, tshape, 0)
                * jnp.uint32(COLS)
                + jax.lax.broadcasted_iota(jnp.uint32, tshape, 1))

    def pass1(c, m):
        off = c * CHUNK
        z = _z_chunk(logits_ref[:, pl.ds(off, CHUNK)],
                     base + jnp.uint32(off).astype(jnp.uint32) + rowcol)
        out_ref[:, pl.ds(off, CHUNK)] = z
        return jnp.maximum(m, jnp.max(z, axis=-1, keepdims=True))

    m0 = jnp.full((ROW_BLOCK, 1), -jnp.inf, dtype=jnp.float32)
    m = jax.lax.fori_loop(0, NFULL, pass1, m0)
    z_t = _z_chunk(logits_ref[:, NFULL * CHUNK:],
                   base + jnp.uint32(NFULL * CHUNK) + rowcol_t)
    out_ref[:, NFULL * CHUNK:] = z_t
    m = jnp.maximum(m, jnp.max(z_t, axis=-1, keepdims=True))

    def pass2(c, s):
        off = c * CHUNK
        e = jnp.exp(out_ref[:, pl.ds(off, CHUNK)] - m)
        out_ref[:, pl.ds(off, CHUNK)] = e
        return s + jnp.sum(e, axis=-1, keepdims=True)

    s0 = jnp.zeros((ROW_BLOCK, 1), dtype=jnp.float32)
    s = jax.lax.fori_loop(0, NFULL, pass2, s0)
    e_t = jnp.exp(out_ref[:, NFULL * CHUNK:] - m)
    out_ref[:, NFULL * CHUNK:] = e_t
    s = s + jnp.sum(e_t, axis=-1, keepdims=True)

    r = jnp.float32(1.0) / s

    def pass3(c, carry):
        off = c * CHUNK
        out_ref[:, pl.ds(off, CHUNK)] = out_ref[:, pl.ds(off, CHUNK)] * r
        return carry

    jax.lax.fori_loop(0, NFULL, pass3, 0)
    out_ref[:, NFULL * CHUNK:] = out_ref[:, NFULL * CHUNK:] * r


@jax.jit
def kernel(logits):
    grid = (ROWS // ROW_BLOCK,)
    return pl.pallas_call(
        _gumbel_softmax_block,
        grid=grid,
        in_specs=[pl.BlockSpec((ROW_BLOCK, COLS), lambda i: (i, 0))],
        out_specs=pl.BlockSpec((ROW_BLOCK, COLS), lambda i: (i, 0)),
        out_shape=jax.ShapeDtypeStruct((ROWS, COLS), jnp.float32),
    )(logits)


# speedup vs baseline: 1.6348x; 1.3425x over previous
import jax
import jax.numpy as jnp
from jax.experimental import pallas as pl

ROWS, COLS = 128, 100000
ROW_BLOCK = 8
CHUNK = 4096
NFULL = COLS // CHUNK
TAIL = COLS - NFULL * CHUNK
LANES = 128


def _threefry_bits_xor(flat_u32):
    k0 = jnp.uint32(0)
    k1 = jnp.uint32(1)
    ks2 = jnp.uint32(0x1BD11BDA) ^ k0 ^ k1
    rot1 = (13, 15, 26, 6)
    rot2 = (17, 29, 16, 24)

    def rotl(x, r):
        return (x << jnp.uint32(r)) | (x >> jnp.uint32(32 - r))

    x0 = jnp.zeros_like(flat_u32) + k0
    x1 = flat_u32 + k1
    inject = ((k1, ks2, 1), (ks2, k0, 2), (k0, k1, 3), (k1, ks2, 4),
              (ks2, k0, 5))
    for i in range(5):
        for r in (rot1 if i % 2 == 0 else rot2):
            x0 = x0 + x1
            x1 = rotl(x1, r)
            x1 = x1 ^ x0
        a, b, c = inject[i]
        x0 = x0 + a
        x1 = x1 + b + jnp.uint32(c)
    return x0 ^ x1


def _z_chunk(logits_chunk, flat_u32):
    bits = _threefry_bits_xor(flat_u32)
    fbits = jax.lax.bitcast_convert_type(
        (bits >> jnp.uint32(9)) | jnp.uint32(0x3F800000), jnp.float32)
    minval = jnp.float32(1e-10)
    u = fbits - jnp.float32(1.0)
    u = u * (jnp.float32(1.0) - minval) + minval
    u = jnp.maximum(u, minval)
    noise = -jnp.log(-jnp.log(u))
    return logits_chunk + noise


def _gumbel_softmax_block(logits_ref, out_ref):
    i = pl.program_id(0)
    base = jnp.uint32(i) * jnp.uint32(ROW_BLOCK * COLS)

    cshape = (ROW_BLOCK, CHUNK)
    rowcol = (jax.lax.broadcasted_iota(jnp.uint32, cshape, 0)
              * jnp.uint32(COLS)
              + jax.lax.broadcasted_iota(jnp.uint32, cshape, 1))
    tshape = (ROW_BLOCK, TAIL)
    rowcol_t = (jax.lax.broadcasted_iota(jnp.uint32, tshape, 0)
                * jnp.uint32(COLS)
                + jax.lax.broadcasted_iota(jnp.uint32, tshape, 1))

    def lanefold(acc, chunk, op):
        for k in range(chunk.shape[1] // LANES):
            acc = op(acc, chunk[:, k * LANES:(k + 1) * LANES])
        return acc

    def pass1(c, m):
        off = c * CHUNK
        z = _z_chunk(logits_ref[:, pl.ds(off, CHUNK)],
                     base + jnp.uint32(off).astype(jnp.uint32) + rowcol)
        out_ref[:, pl.ds(off, CHUNK)] = z
        return lanefold(m, z, jnp.maximum)

    m0 = jnp.full((ROW_BLOCK, LANES), -jnp.inf, dtype=jnp.float32)
    macc = jax.lax.fori_loop(0, NFULL, pass1, m0)
    z_t = _z_chunk(logits_ref[:, NFULL * CHUNK:],
                   base + jnp.uint32(NFULL * CHUNK) + rowcol_t)
    out_ref[:, NFULL * CHUNK:] = z_t
    m = jnp.max(macc, axis=-1, keepdims=True)
    m = jnp.maximum(m, jnp.max(z_t, axis=-1, keepdims=True))

    def pass2(c, s):
        off = c * CHUNK
        e = jnp.exp(out_ref[:, pl.ds(off, CHUNK)] - m)
        out_ref[:, pl.ds(off, CHUNK)] = e
        return lanefold(s, e, jnp.add)

    s0 = jnp.zeros((ROW_BLOCK, LANES), dtype=jnp.float32)
    sacc = jax.lax.fori_loop(0, NFULL, pass2, s0)
    e_t = jnp.exp(out_ref[:, NFULL * CHUNK:] - m)
    out_ref[:, NFULL * CHUNK:] = e_t
    s = jnp.sum(sacc, axis=-1, keepdims=True)
    s = s + jnp.sum(e_t, axis=-1, keepdims=True)

    r = jnp.float32(1.0) / s

    def pass3(c, carry):
        off = c * CHUNK
        out_ref[:, pl.ds(off, CHUNK)] = out_ref[:, pl.ds(off, CHUNK)] * r
        return carry

    jax.lax.fori_loop(0, NFULL, pass3, 0)
    out_ref[:, NFULL * CHUNK:] = out_ref[:, NFULL * CHUNK:] * r


@jax.jit
def kernel(logits):
    grid = (ROWS // ROW_BLOCK,)
    return pl.pallas_call(
        _gumbel_softmax_block,
        grid=grid,
        in_specs=[pl.BlockSpec((ROW_BLOCK, COLS), lambda i: (i, 0))],
        out_specs=pl.BlockSpec((ROW_BLOCK, COLS), lambda i: (i, 0)),
        out_shape=jax.ShapeDtypeStruct((ROWS, COLS), jnp.float32),
    )(logits)


# speedup vs baseline: 1.6478x; 1.0080x over previous
import jax
import jax.numpy as jnp
from jax.experimental import pallas as pl
from jax.experimental.pallas import tpu as pltpu

ROWS, COLS = 128, 100000
ROW_BLOCK = 8
CHUNK = 4096
NFULL = COLS // CHUNK
TAIL = COLS - NFULL * CHUNK
LANES = 128


def _threefry_bits_xor(flat_u32):
    k0 = jnp.uint32(0)
    k1 = jnp.uint32(1)
    ks2 = jnp.uint32(0x1BD11BDA) ^ k0 ^ k1
    rot1 = (13, 15, 26, 6)
    rot2 = (17, 29, 16, 24)

    def rotl(x, r):
        return (x << jnp.uint32(r)) | (x >> jnp.uint32(32 - r))

    x0 = jnp.zeros_like(flat_u32) + k0
    x1 = flat_u32 + k1
    inject = ((k1, ks2, 1), (ks2, k0, 2), (k0, k1, 3), (k1, ks2, 4),
              (ks2, k0, 5))
    for i in range(5):
        for r in (rot1 if i % 2 == 0 else rot2):
            x0 = x0 + x1
            x1 = rotl(x1, r)
            x1 = x1 ^ x0
        a, b, c = inject[i]
        x0 = x0 + a
        x1 = x1 + b + jnp.uint32(c)
    return x0 ^ x1


def _z_chunk(logits_chunk, flat_u32):
    bits = _threefry_bits_xor(flat_u32)
    fbits = jax.lax.bitcast_convert_type(
        (bits >> jnp.uint32(9)) | jnp.uint32(0x3F800000), jnp.float32)
    minval = jnp.float32(1e-10)
    u = fbits - jnp.float32(1.0)
    u = u * (jnp.float32(1.0) - minval) + minval
    u = jnp.maximum(u, minval)
    noise = -jnp.log(-jnp.log(u))
    return logits_chunk + noise


def _gumbel_softmax_block(logits_ref, out_ref, z_ref):
    i = pl.program_id(0)
    base = jnp.uint32(i) * jnp.uint32(ROW_BLOCK * COLS)

    cshape = (ROW_BLOCK, CHUNK)
    rowcol = (jax.lax.broadcasted_iota(jnp.uint32, cshape, 0)
              * jnp.uint32(COLS)
              + jax.lax.broadcasted_iota(jnp.uint32, cshape, 1))
    tshape = (ROW_BLOCK, TAIL)
    rowcol_t = (jax.lax.broadcasted_iota(jnp.uint32, tshape, 0)
                * jnp.uint32(COLS)
                + jax.lax.broadcasted_iota(jnp.uint32, tshape, 1))

    def lanefold(acc, chunk, op):
        for k in range(chunk.shape[1] // LANES):
            acc = op(acc, chunk[:, k * LANES:(k + 1) * LANES])
        return acc

    def pass1(c, m):
        off = c * CHUNK
        z = _z_chunk(logits_ref[:, pl.ds(off, CHUNK)],
                     base + jnp.uint32(off).astype(jnp.uint32) + rowcol)
        z_ref[:, pl.ds(off, CHUNK)] = z
        return lanefold(m, z, jnp.maximum)

    m0 = jnp.full((ROW_BLOCK, LANES), -jnp.inf, dtype=jnp.float32)
    macc = jax.lax.fori_loop(0, NFULL, pass1, m0)
    z_t = _z_chunk(logits_ref[:, NFULL * CHUNK:],
                   base + jnp.uint32(NFULL * CHUNK) + rowcol_t)
    z_ref[:, NFULL * CHUNK:] = z_t
    m = jnp.max(macc, axis=-1, keepdims=True)
    m = jnp.maximum(m, jnp.max(z_t, axis=-1, keepdims=True))

    def pass2(c, s):
        off = c * CHUNK
        e = jnp.exp(z_ref[:, pl.ds(off, CHUNK)] - m)
        z_ref[:, pl.ds(off, CHUNK)] = e
        return lanefold(s, e, jnp.add)

    s0 = jnp.zeros((ROW_BLOCK, LANES), dtype=jnp.float32)
    sacc = jax.lax.fori_loop(0, NFULL, pass2, s0)
    e_t = jnp.exp(z_ref[:, NFULL * CHUNK:] - m)
    z_ref[:, NFULL * CHUNK:] = e_t
    s = jnp.sum(sacc, axis=-1, keepdims=True)
    s = s + jnp.sum(e_t, axis=-1, keepdims=True)

    r = jnp.float32(1.0) / s

    def pass3(c, carry):
        off = c * CHUNK
        out_ref[:, pl.ds(off, CHUNK)] = z_ref[:, pl.ds(off, CHUNK)] * r
        return carry

    jax.lax.fori_loop(0, NFULL, pass3, 0)
    out_ref[:, NFULL * CHUNK:] = z_ref[:, NFULL * CHUNK:] * r


@jax.jit
def kernel(logits):
    grid = (ROWS // ROW_BLOCK,)
    return pl.pallas_call(
        _gumbel_softmax_block,
        grid=grid,
        in_specs=[pl.BlockSpec((ROW_BLOCK, COLS), lambda i: (i, 0))],
        out_specs=pl.BlockSpec((ROW_BLOCK, COLS), lambda i: (i, 0)),
        out_shape=jax.ShapeDtypeStruct((ROWS, COLS), jnp.float32),
        scratch_shapes=[pltpu.VMEM((ROW_BLOCK, COLS), jnp.float32)],
    )(logits)


# speedup vs baseline: 1.7054x; 1.0349x over previous
import jax
import jax.numpy as jnp
from jax.experimental import pallas as pl
from jax.experimental.pallas import tpu as pltpu

ROWS, COLS = 128, 100000
ROW_BLOCK = 16
CHUNK = 4096
NFULL = COLS // CHUNK
TAIL = COLS - NFULL * CHUNK
LANES = 128


def _threefry_bits_xor(flat_u32):
    k0 = jnp.uint32(0)
    k1 = jnp.uint32(1)
    ks2 = jnp.uint32(0x1BD11BDA) ^ k0 ^ k1
    rot1 = (13, 15, 26, 6)
    rot2 = (17, 29, 16, 24)

    def rotl(x, r):
        return (x << jnp.uint32(r)) | (x >> jnp.uint32(32 - r))

    x0 = jnp.zeros_like(flat_u32) + k0
    x1 = flat_u32 + k1
    inject = ((k1, ks2, 1), (ks2, k0, 2), (k0, k1, 3), (k1, ks2, 4),
              (ks2, k0, 5))
    for i in range(5):
        for r in (rot1 if i % 2 == 0 else rot2):
            x0 = x0 + x1
            x1 = rotl(x1, r)
            x1 = x1 ^ x0
        a, b, c = inject[i]
        x0 = x0 + a
        x1 = x1 + b + jnp.uint32(c)
    return x0 ^ x1


def _z_chunk(logits_chunk, flat_u32):
    bits = _threefry_bits_xor(flat_u32)
    fbits = jax.lax.bitcast_convert_type(
        (bits >> jnp.uint32(9)) | jnp.uint32(0x3F800000), jnp.float32)
    minval = jnp.float32(1e-10)
    u = fbits - jnp.float32(1.0)
    u = u * (jnp.float32(1.0) - minval) + minval
    u = jnp.maximum(u, minval)
    noise = -jnp.log(-jnp.log(u))
    return logits_chunk + noise


def _gumbel_softmax_block(logits_ref, out_ref, z_ref):
    i = pl.program_id(0)
    base = jnp.uint32(i) * jnp.uint32(ROW_BLOCK * COLS)

    cshape = (ROW_BLOCK, CHUNK)
    rowcol = (jax.lax.broadcasted_iota(jnp.uint32, cshape, 0)
              * jnp.uint32(COLS)
              + jax.lax.broadcasted_iota(jnp.uint32, cshape, 1))
    tshape = (ROW_BLOCK, TAIL)
    rowcol_t = (jax.lax.broadcasted_iota(jnp.uint32, tshape, 0)
                * jnp.uint32(COLS)
                + jax.lax.broadcasted_iota(jnp.uint32, tshape, 1))

    def lanefold(acc, chunk, op):
        for k in range(chunk.shape[1] // LANES):
            acc = op(acc, chunk[:, k * LANES:(k + 1) * LANES])
        return acc

    def pass1(c, m):
        off = c * CHUNK
        z = _z_chunk(logits_ref[:, pl.ds(off, CHUNK)],
                     base + jnp.uint32(off).astype(jnp.uint32) + rowcol)
        z_ref[:, pl.ds(off, CHUNK)] = z
        return lanefold(m, z, jnp.maximum)

    m0 = jnp.full((ROW_BLOCK, LANES), -jnp.inf, dtype=jnp.float32)
    macc = jax.lax.fori_loop(0, NFULL, pass1, m0)
    z_t = _z_chunk(logits_ref[:, NFULL * CHUNK:],
                   base + jnp.uint32(NFULL * CHUNK) + rowcol_t)
    z_ref[:, NFULL * CHUNK:] = z_t
    m = jnp.max(macc, axis=-1, keepdims=True)
    m = jnp.maximum(m, jnp.max(z_t, axis=-1, keepdims=True))

    def pass2(c, s):
        off = c * CHUNK
        e = jnp.exp(z_ref[:, pl.ds(off, CHUNK)] - m)
        z_ref[:, pl.ds(off, CHUNK)] = e
        return lanefold(s, e, jnp.add)

    s0 = jnp.zeros((ROW_BLOCK, LANES), dtype=jnp.float32)
    sacc = jax.lax.fori_loop(0, NFULL, pass2, s0)
    e_t = jnp.exp(z_ref[:, NFULL * CHUNK:] - m)
    z_ref[:, NFULL * CHUNK:] = e_t
    s = jnp.sum(sacc, axis=-1, keepdims=True)
    s = s + jnp.sum(e_t, axis=-1, keepdims=True)

    r = jnp.float32(1.0) / s

    def pass3(c, carry):
        off = c * CHUNK
        out_ref[:, pl.ds(off, CHUNK)] = z_ref[:, pl.ds(off, CHUNK)] * r
        return carry

    jax.lax.fori_loop(0, NFULL, pass3, 0)
    out_ref[:, NFULL * CHUNK:] = z_ref[:, NFULL * CHUNK:] * r


@jax.jit
def kernel(logits):
    grid = (ROWS // ROW_BLOCK,)
    return pl.pallas_call(
        _gumbel_softmax_block,
        grid=grid,
        in_specs=[pl.BlockSpec((ROW_BLOCK, COLS), lambda i: (i, 0))],
        out_specs=pl.BlockSpec((ROW_BLOCK, COLS), lambda i: (i, 0)),
        out_shape=jax.ShapeDtypeStruct((ROWS, COLS), jnp.float32),
        scratch_shapes=[pltpu.VMEM((ROW_BLOCK, COLS), jnp.float32)],
    )(logits)
